# baseline (speedup 1.0000x reference)
import functools

import jax
import jax.numpy as jnp
from jax.experimental import pallas as pl

_NUM_EXPERTS = 64
_TOP_K = 8
_EMBED_DIM = 256

_BLOCK_M = 256


def _router_kernel(x_ref, w_ref, e_ref, t_ref,
                   weights_ref, idx_ref, logits_ref, cos_ref, raw_ref):
    proj = jnp.dot(x_ref[...], w_ref[...], preferred_element_type=jnp.float32)

    norm = jnp.sqrt(jnp.sum(proj * proj, axis=1, keepdims=True))
    proj_n = proj / jnp.maximum(norm, 1e-12)

    emb = e_ref[...]
    enorm = jnp.sqrt(jnp.sum(emb * emb, axis=0, keepdims=True))
    emb_n = emb / jnp.maximum(enorm, 1e-12)

    cos = jnp.dot(proj_n, emb_n, preferred_element_type=jnp.float32)
    logits = cos * t_ref[0, 0]

    cos_ref[...] = cos
    logits_ref[...] = logits

    bm = logits.shape[0]
    iota = jax.lax.broadcasted_iota(jnp.int32, (bm, _NUM_EXPERTS), 1)
    work = logits
    mask = jnp.zeros((bm, _NUM_EXPERTS), dtype=jnp.bool_)
    for j in range(_TOP_K):
        m = jnp.max(work, axis=1, keepdims=True)
        cand = jnp.where(work == m, iota, _NUM_EXPERTS)
        idx = jnp.min(cand, axis=1, keepdims=True)
        sel = iota == idx
        mask = jnp.logical_or(mask, sel)
        work = jnp.where(sel, -jnp.inf, work)
        idx_ref[:, j] = idx[:, 0]

    masked = jnp.where(mask, logits, -1e9)
    mmax = jnp.max(masked, axis=1, keepdims=True)
    mexp = jnp.exp(masked - mmax)
    weights_ref[...] = mexp / jnp.sum(mexp, axis=1, keepdims=True)

    rmax = jnp.max(logits, axis=1, keepdims=True)
    rexp = jnp.exp(logits - rmax)
    raw_ref[...] = rexp / jnp.sum(rexp, axis=1, keepdims=True)


@functools.partial(jax.jit, static_argnames=("interpret",))
def kernel(inputs, W_proj, expert_embeddings, temperature, interpret=False):
    tokens, d_model = inputs.shape
    grid = (tokens // _BLOCK_M,)
    temp = jnp.reshape(temperature.astype(jnp.float32), (1, 1))

    out_shapes = (
        jax.ShapeDtypeStruct((tokens, _NUM_EXPERTS), jnp.float32),
        jax.ShapeDtypeStruct((tokens, _TOP_K), jnp.int32),
        jax.ShapeDtypeStruct((tokens, _NUM_EXPERTS), jnp.float32),
        jax.ShapeDtypeStruct((tokens, _NUM_EXPERTS), jnp.float32),
        jax.ShapeDtypeStruct((tokens, _NUM_EXPERTS), jnp.float32),
    )

    row_spec = pl.BlockSpec((_BLOCK_M, _NUM_EXPERTS), lambda i: (i, 0))
    outs = pl.pallas_call(
        _router_kernel,
        grid=grid,
        in_specs=[
            pl.BlockSpec((_BLOCK_M, d_model), lambda i: (i, 0)),
            pl.BlockSpec((d_model, _EMBED_DIM), lambda i: (0, 0)),
            pl.BlockSpec((_EMBED_DIM, _NUM_EXPERTS), lambda i: (0, 0)),
            pl.BlockSpec((1, 1), lambda i: (0, 0)),
        ],
        out_specs=(
            row_spec,
            pl.BlockSpec((_BLOCK_M, _TOP_K), lambda i: (i, 0)),
            row_spec,
            row_spec,
            row_spec,
        ),
        out_shape=out_shapes,
        interpret=interpret,
    )(inputs, W_proj, expert_embeddings, temp)

    return outs


# speedup vs baseline: 1.2110x; 1.2110x over previous
import functools

import jax
import jax.numpy as jnp
from jax.experimental import pallas as pl

_NUM_EXPERTS = 64
_TOP_K = 8
_EMBED_DIM = 256

_BLOCK_M = 256


def _router_kernel(x_ref, w_ref, e_ref, t_ref,
                   weights_ref, idx_ref, logits_ref, cos_ref, raw_ref):
    proj = jnp.dot(x_ref[...], w_ref[...], preferred_element_type=jnp.float32)

    norm = jnp.sqrt(jnp.sum(proj * proj, axis=1, keepdims=True))
    proj_n = proj / jnp.maximum(norm, 1e-12)

    emb = e_ref[...]
    enorm = jnp.sqrt(jnp.sum(emb * emb, axis=0, keepdims=True))
    emb_n = emb / jnp.maximum(enorm, 1e-12)

    cos = jnp.dot(proj_n, emb_n, preferred_element_type=jnp.float32)
    logits = cos * t_ref[0, 0]

    cos_ref[...] = cos
    logits_ref[...] = logits

    bm = logits.shape[0]
    iota_i = jax.lax.broadcasted_iota(jnp.int32, (bm, _NUM_EXPERTS), 1)
    riota = 63.0 - iota_i.astype(jnp.float32)
    rmax = jnp.max(logits, axis=1, keepdims=True)
    work = logits
    mask = jnp.zeros((bm, _NUM_EXPERTS), dtype=jnp.bool_)
    rs = []
    m = rmax
    for j in range(_TOP_K):
        cand = jnp.where(work == m, riota, -1.0)
        r = jnp.max(cand, axis=1, keepdims=True)
        sel = cand == r
        mask = jnp.logical_or(mask, sel)
        rs.append(r)
        if j + 1 < _TOP_K:
            work = jnp.where(sel, -jnp.inf, work)
            m = jnp.max(work, axis=1, keepdims=True)
    idx_ref[...] = (63.0 - jnp.concatenate(rs, axis=1)).astype(jnp.int32)

    rexp = jnp.exp(logits - rmax)
    raw_ref[...] = rexp / jnp.sum(rexp, axis=1, keepdims=True)
    mexp = jnp.where(mask, rexp, 0.0)
    weights_ref[...] = mexp / jnp.sum(mexp, axis=1, keepdims=True)


@functools.partial(jax.jit, static_argnames=("interpret",))
def kernel(inputs, W_proj, expert_embeddings, temperature, interpret=False):
    tokens, d_model = inputs.shape
    grid = (tokens // _BLOCK_M,)
    temp = jnp.reshape(temperature.astype(jnp.float32), (1, 1))

    out_shapes = (
        jax.ShapeDtypeStruct((tokens, _NUM_EXPERTS), jnp.float32),
        jax.ShapeDtypeStruct((tokens, _TOP_K), jnp.int32),
        jax.ShapeDtypeStruct((tokens, _NUM_EXPERTS), jnp.float32),
        jax.ShapeDtypeStruct((tokens, _NUM_EXPERTS), jnp.float32),
        jax.ShapeDtypeStruct((tokens, _NUM_EXPERTS), jnp.float32),
    )

    row_spec = pl.BlockSpec((_BLOCK_M, _NUM_EXPERTS), lambda i: (i, 0))
    outs = pl.pallas_call(
        _router_kernel,
        grid=grid,
        in_specs=[
            pl.BlockSpec((_BLOCK_M, d_model), lambda i: (i, 0)),
            pl.BlockSpec((d_model, _EMBED_DIM), lambda i: (0, 0)),
            pl.BlockSpec((_EMBED_DIM, _NUM_EXPERTS), lambda i: (0, 0)),
            pl.BlockSpec((1, 1), lambda i: (0, 0)),
        ],
        out_specs=(
            row_spec,
            pl.BlockSpec((_BLOCK_M, _TOP_K), lambda i: (i, 0)),
            row_spec,
            row_spec,
            row_spec,
        ),
        out_shape=out_shapes,
        interpret=interpret,
    )(inputs, W_proj, expert_embeddings, temp)

    return outs


# speedup vs baseline: 1.5621x; 1.2899x over previous
import functools

import jax
import jax.numpy as jnp
from jax.experimental import pallas as pl
from jax.experimental.pallas import tpu as pltpu

_NUM_EXPERTS = 64
_TOP_K = 8
_EMBED_DIM = 256

_BLOCK_M = 512


def _router_kernel(x_ref, w_ref, e_ref, t_ref,
                   weights_ref, idx_ref, logits_ref, cos_ref, raw_ref):
    proj = jnp.dot(x_ref[...], w_ref[...], preferred_element_type=jnp.float32)

    norm = jnp.sqrt(jnp.sum(proj * proj, axis=1, keepdims=True))
    proj_n = proj / jnp.maximum(norm, 1e-12)

    emb = e_ref[...]
    enorm = jnp.sqrt(jnp.sum(emb * emb, axis=0, keepdims=True))
    emb_n = emb / jnp.maximum(enorm, 1e-12)

    cos = jnp.dot(proj_n, emb_n, preferred_element_type=jnp.float32)
    logits = cos * t_ref[0, 0]

    cos_ref[...] = cos
    logits_ref[...] = logits

    bm = logits.shape[0]
    iota_i = jax.lax.broadcasted_iota(jnp.int32, (bm, _NUM_EXPERTS), 1)
    riota = 63.0 - iota_i.astype(jnp.float32)
    rmax = jnp.max(logits, axis=1, keepdims=True)
    work = logits
    mask = jnp.zeros((bm, _NUM_EXPERTS), dtype=jnp.bool_)
    rs = []
    m = rmax
    for j in range(_TOP_K):
        cand = jnp.where(work == m, riota, -1.0)
        r = jnp.max(cand, axis=1, keepdims=True)
        sel = cand == r
        mask = jnp.logical_or(mask, sel)
        rs.append(r)
        if j + 1 < _TOP_K:
            work = jnp.where(sel, -jnp.inf, work)
            m = jnp.max(work, axis=1, keepdims=True)
    idx_ref[...] = (63.0 - jnp.concatenate(rs, axis=1)).astype(jnp.int32)

    rexp = jnp.exp(logits - rmax)
    raw_ref[...] = rexp / jnp.sum(rexp, axis=1, keepdims=True)
    mexp = jnp.where(mask, rexp, 0.0)
    weights_ref[...] = mexp / jnp.sum(mexp, axis=1, keepdims=True)


@functools.partial(jax.jit, static_argnames=("interpret",))
def kernel(inputs, W_proj, expert_embeddings, temperature, interpret=False):
    tokens, d_model = inputs.shape
    grid = (tokens // _BLOCK_M,)
    temp = jnp.reshape(temperature.astype(jnp.float32), (1, 1))

    out_shapes = (
        jax.ShapeDtypeStruct((tokens, _NUM_EXPERTS), jnp.float32),
        jax.ShapeDtypeStruct((tokens, _TOP_K), jnp.int32),
        jax.ShapeDtypeStruct((tokens, _NUM_EXPERTS), jnp.float32),
        jax.ShapeDtypeStruct((tokens, _NUM_EXPERTS), jnp.float32),
        jax.ShapeDtypeStruct((tokens, _NUM_EXPERTS), jnp.float32),
    )

    row_spec = pl.BlockSpec((_BLOCK_M, _NUM_EXPERTS), lambda i: (i, 0))
    outs = pl.pallas_call(
        _router_kernel,
        grid=grid,
        in_specs=[
            pl.BlockSpec((_BLOCK_M, d_model), lambda i: (i, 0)),
            pl.BlockSpec((d_model, _EMBED_DIM), lambda i: (0, 0)),
            pl.BlockSpec((_EMBED_DIM, _NUM_EXPERTS), lambda i: (0, 0)),
            pl.BlockSpec((1, 1), lambda i: (0, 0)),
        ],
        out_specs=(
            row_spec,
            pl.BlockSpec((_BLOCK_M, _TOP_K), lambda i: (i, 0)),
            row_spec,
            row_spec,
            row_spec,
        ),
        out_shape=out_shapes,
        compiler_params=pltpu.CompilerParams(
            dimension_semantics=("arbitrary",),
        ),
        interpret=interpret,
    )(inputs, W_proj, expert_embeddings, temp)

    return outs


# speedup vs baseline: 1.7418x; 1.1150x over previous
import functools

import jax
import jax.numpy as jnp
from jax.experimental import pallas as pl
from jax.experimental.pallas import tpu as pltpu

_NUM_EXPERTS = 64
_TOP_K = 8
_EMBED_DIM = 256

_BLOCK_M = 1024


def _router_kernel(x_ref, w_ref, e_ref, t_ref,
                   weights_ref, idx_ref, logits_ref, cos_ref, raw_ref):
    proj = jnp.dot(x_ref[...], w_ref[...], preferred_element_type=jnp.float32)

    norm = jnp.sqrt(jnp.sum(proj * proj, axis=1, keepdims=True))
    proj_n = proj / jnp.maximum(norm, 1e-12)

    emb = e_ref[...]
    enorm = jnp.sqrt(jnp.sum(emb * emb, axis=0, keepdims=True))
    emb_n = emb / jnp.maximum(enorm, 1e-12)

    cos = jnp.dot(proj_n, emb_n, preferred_element_type=jnp.float32)
    logits = cos * t_ref[0, 0]

    cos_ref[...] = cos
    logits_ref[...] = logits

    bm = logits.shape[0]
    iota_i = jax.lax.broadcasted_iota(jnp.int32, (bm, _NUM_EXPERTS), 1)
    riota = 63.0 - iota_i.astype(jnp.float32)
    rmax = jnp.max(logits, axis=1, keepdims=True)
    work = logits
    mask = jnp.zeros((bm, _NUM_EXPERTS), dtype=jnp.bool_)
    rs = []
    m = rmax
    for j in range(_TOP_K):
        cand = jnp.where(work == m, riota, -1.0)
        r = jnp.max(cand, axis=1, keepdims=True)
        sel = cand == r
        mask = jnp.logical_or(mask, sel)
        rs.append(r)
        if j + 1 < _TOP_K:
            work = jnp.where(sel, -jnp.inf, work)
            m = jnp.max(work, axis=1, keepdims=True)
    idx_ref[...] = (63.0 - jnp.concatenate(rs, axis=1)).astype(jnp.int32)

    rexp = jnp.exp(logits - rmax)
    raw_ref[...] = rexp / jnp.sum(rexp, axis=1, keepdims=True)
    mexp = jnp.where(mask, rexp, 0.0)
    weights_ref[...] = mexp / jnp.sum(mexp, axis=1, keepdims=True)


@functools.partial(jax.jit, static_argnames=("interpret",))
def kernel(inputs, W_proj, expert_embeddings, temperature, interpret=False):
    tokens, d_model = inputs.shape
    grid = (tokens // _BLOCK_M,)
    temp = jnp.reshape(temperature.astype(jnp.float32), (1, 1))

    out_shapes = (
        jax.ShapeDtypeStruct((tokens, _NUM_EXPERTS), jnp.float32),
        jax.ShapeDtypeStruct((tokens, _TOP_K), jnp.int32),
        jax.ShapeDtypeStruct((tokens, _NUM_EXPERTS), jnp.float32),
        jax.ShapeDtypeStruct((tokens, _NUM_EXPERTS), jnp.float32),
        jax.ShapeDtypeStruct((tokens, _NUM_EXPERTS), jnp.float32),
    )

    row_spec = pl.BlockSpec((_BLOCK_M, _NUM_EXPERTS), lambda i: (i, 0))
    outs = pl.pallas_call(
        _router_kernel,
        grid=grid,
        in_specs=[
            pl.BlockSpec((_BLOCK_M, d_model), lambda i: (i, 0)),
            pl.BlockSpec((d_model, _EMBED_DIM), lambda i: (0, 0)),
            pl.BlockSpec((_EMBED_DIM, _NUM_EXPERTS), lambda i: (0, 0)),
            pl.BlockSpec((1, 1), lambda i: (0, 0)),
        ],
        out_specs=(
            row_spec,
            pl.BlockSpec((_BLOCK_M, _TOP_K), lambda i: (i, 0)),
            row_spec,
            row_spec,
            row_spec,
        ),
        out_shape=out_shapes,
        compiler_params=pltpu.CompilerParams(
            dimension_semantics=("arbitrary",),
        ),
        interpret=interpret,
    )(inputs, W_proj, expert_embeddings, temp)

    return outs
